# baseline (speedup 1.0000x reference)
import functools

import jax
import jax.numpy as jnp
from jax import lax
from jax.experimental import pallas as pl
from jax.experimental.pallas import tpu as pltpu
from jax.experimental.pallas import tpu_sc as plsc

N_NODES = 100000
D = 128
NUM_TYPES = 64

_info = plsc.get_sparse_core_info()
NC, NS = _info.num_cores, _info.num_subcores
NW = NC * NS

MAIN = (N_NODES // (8 * NW)) * (8 * NW)
B_W = MAIN // NW
CHUNK = 104
N_CHUNKS = B_W // CHUNK
NBUF = 5
N_ROUNDS = N_CHUNKS // NBUF
TAIL = N_NODES - MAIN
TAIL_STEP = 8
TAIL_W = TAIL // TAIL_STEP

_mesh = plsc.VectorSubcoreMesh(core_axis_name="c", subcore_axis_name="s")


@functools.partial(
    pl.kernel,
    mesh=_mesh,
    out_type=jax.ShapeDtypeStruct((N_NODES, D), jnp.float32),
    scratch_types=[
        pltpu.VMEM((B_W,), jnp.int32),
        pltpu.VMEM((TAIL_STEP,), jnp.int32),
    ] + [pltpu.VMEM((CHUNK, D), jnp.float32) for _ in range(NBUF)] + [
        pltpu.VMEM_SHARED((NUM_TYPES, D), jnp.float32),
    ] + [pltpu.SemaphoreType.DMA for _ in range(2 * NBUF)],
)
def _embed_sc(idx_hbm, table_hbm, out_hbm, idx_v, tidx_v, *rest):
    bufs = rest[:NBUF]
    table_sh = rest[NBUF]
    gsem = rest[NBUF + 1:NBUF + 1 + NBUF]
    ssem = rest[NBUF + 1 + NBUF:]
    wid = lax.axis_index("s") * NC + lax.axis_index("c")
    base = wid * B_W

    @pl.when(lax.axis_index("s") == 0)
    def _stage():
        pltpu.sync_copy(table_hbm, table_sh)
    plsc.subcore_barrier()

    pltpu.sync_copy(idx_hbm.at[pl.ds(base, B_W)], idx_v)

    def gather_start(c, b):
        pltpu.async_copy(
            table_sh.at[idx_v.at[pl.ds(c * CHUNK, CHUNK)]], bufs[b], gsem[b]
        )

    def gather_wait(b):
        pltpu.make_async_copy(
            table_sh.at[idx_v.at[pl.ds(0, CHUNK)]], bufs[b], gsem[b]
        ).wait()

    def store_start(c, b):
        pltpu.async_copy(
            bufs[b], out_hbm.at[pl.ds(base + c * CHUNK, CHUNK)], ssem[b]
        )

    def store_wait(b):
        pltpu.make_async_copy(
            bufs[b], out_hbm.at[pl.ds(base, CHUNK)], ssem[b]
        ).wait()

    def body(r, carry):
        for b in range(NBUF):
            @pl.when(r > 0)
            def _(b=b):
                store_wait(b)
            gather_start(r * NBUF + b, b)
        for b in range(NBUF):
            gather_wait(b)
            store_start(r * NBUF + b, b)
        return carry

    lax.fori_loop(0, N_ROUNDS, body, 0)
    for b in range(NBUF):
        store_wait(b)

    @pl.when(wid < TAIL_W)
    def _tail():
        tb = MAIN + wid * TAIL_STEP
        pltpu.sync_copy(idx_hbm.at[pl.ds(tb, TAIL_STEP)], tidx_v)
        pltpu.async_copy(
            table_sh.at[tidx_v], bufs[0].at[pl.ds(0, TAIL_STEP)], gsem[0]
        ).wait()
        pltpu.sync_copy(
            bufs[0].at[pl.ds(0, TAIL_STEP)], out_hbm.at[pl.ds(tb, TAIL_STEP)]
        )


TC_BLOCK = 1000
TC_GRID = N_NODES // TC_BLOCK


def _tc_body(idx_ref, table_ref, out_ref):
    idx_col = idx_ref[0].reshape(TC_BLOCK, 1)
    tids = lax.broadcasted_iota(jnp.int32, (TC_BLOCK, NUM_TYPES), 1)
    onehot = (idx_col == tids).astype(jnp.float32)
    out_ref[...] = jnp.dot(
        onehot, table_ref[...], preferred_element_type=jnp.float32
    )


_embed_tc = pl.pallas_call(
    _tc_body,
    grid=(TC_GRID,),
    in_specs=[
        pl.BlockSpec((1, 1, TC_BLOCK), lambda i: (i, 0, 0)),
        pl.BlockSpec((NUM_TYPES, D), lambda i: (0, 0)),
    ],
    out_specs=pl.BlockSpec((TC_BLOCK, D), lambda i: (i, 0)),
    out_shape=jax.ShapeDtypeStruct((N_NODES, D), jnp.float32),
)


def kernel(atom_types, embed_table):
    idx = atom_types.reshape(-1).astype(jnp.int32)
    out0 = _embed_sc(idx, embed_table)
    out1 = _embed_tc(idx.reshape(TC_GRID, 1, TC_BLOCK), embed_table)
    return (out0, out1)


# speedup vs baseline: 1.3795x; 1.3795x over previous
import functools

import jax
import jax.numpy as jnp
from jax import lax
from jax.experimental import pallas as pl
from jax.experimental.pallas import tpu as pltpu
from jax.experimental.pallas import tpu_sc as plsc

N_NODES = 100000
D = 128
NUM_TYPES = 64

_info = plsc.get_sparse_core_info()
NC, NS = _info.num_cores, _info.num_subcores
NW = NC * NS

MAIN = (N_NODES // (8 * NW)) * (8 * NW)
B_W = MAIN // NW
CHUNK = 104
N_CHUNKS = B_W // CHUNK
NBUF = 5
N_ROUNDS = N_CHUNKS // NBUF
TAIL = N_NODES - MAIN
TAIL_STEP = 8
TAIL_W = TAIL // TAIL_STEP

_mesh = plsc.VectorSubcoreMesh(core_axis_name="c", subcore_axis_name="s")


@functools.partial(
    pl.kernel,
    mesh=_mesh,
    out_type=jax.ShapeDtypeStruct((N_NODES, D), jnp.float32),
    scratch_types=[
        pltpu.VMEM((B_W,), jnp.int32),
        pltpu.VMEM((TAIL_STEP,), jnp.int32),
    ] + [pltpu.VMEM((CHUNK, D), jnp.float32) for _ in range(NBUF)] + [
        pltpu.VMEM_SHARED((NUM_TYPES, D), jnp.float32),
    ] + [pltpu.SemaphoreType.DMA for _ in range(2 * NBUF)],
)
def _embed_sc(idx_hbm, table_hbm, out_hbm, idx_v, tidx_v, *rest):
    bufs = rest[:NBUF]
    table_sh = rest[NBUF]
    gsem = rest[NBUF + 1:NBUF + 1 + NBUF]
    ssem = rest[NBUF + 1 + NBUF:]
    wid = lax.axis_index("s") * NC + lax.axis_index("c")
    base = wid * B_W

    @pl.when(lax.axis_index("s") == 0)
    def _stage():
        pltpu.sync_copy(table_hbm, table_sh)
    plsc.subcore_barrier()

    pltpu.sync_copy(idx_hbm.at[pl.ds(base, B_W)], idx_v)

    def gather_start(c, b):
        pltpu.async_copy(
            table_sh.at[idx_v.at[pl.ds(c * CHUNK, CHUNK)]], bufs[b], gsem[b]
        )

    def gather_wait(b):
        pltpu.make_async_copy(
            table_sh.at[idx_v.at[pl.ds(0, CHUNK)]], bufs[b], gsem[b]
        ).wait()

    def store_start(c, b):
        pltpu.async_copy(
            bufs[b], out_hbm.at[pl.ds(base + c * CHUNK, CHUNK)], ssem[b]
        )

    def store_wait(b):
        pltpu.make_async_copy(
            bufs[b], out_hbm.at[pl.ds(base, CHUNK)], ssem[b]
        ).wait()

    def body(r, carry):
        for b in range(NBUF):
            @pl.when(r > 0)
            def _(b=b):
                store_wait(b)
            gather_start(r * NBUF + b, b)
        for b in range(NBUF):
            gather_wait(b)
            store_start(r * NBUF + b, b)
        return carry

    lax.fori_loop(0, N_ROUNDS, body, 0)
    for b in range(NBUF):
        store_wait(b)

    @pl.when(wid < TAIL_W)
    def _tail():
        tb = MAIN + wid * TAIL_STEP
        pltpu.sync_copy(idx_hbm.at[pl.ds(tb, TAIL_STEP)], tidx_v)
        pltpu.async_copy(
            table_sh.at[tidx_v], bufs[0].at[pl.ds(0, TAIL_STEP)], gsem[0]
        ).wait()
        pltpu.sync_copy(
            bufs[0].at[pl.ds(0, TAIL_STEP)], out_hbm.at[pl.ds(tb, TAIL_STEP)]
        )


TC_BLOCK = 2000
TC_GRID = N_NODES // TC_BLOCK


def _tc_body(idx_ref, table_ref, out_ref):
    idx_row = idx_ref[0]
    tids = lax.broadcasted_iota(jnp.int32, (NUM_TYPES, TC_BLOCK), 0)
    onehot_t = (idx_row == tids).astype(jnp.float32)
    out_ref[...] = lax.dot_general(
        onehot_t, table_ref[...], (((0,), (0,)), ((), ())),
        preferred_element_type=jnp.float32,
    )


_embed_tc = pl.pallas_call(
    _tc_body,
    grid=(TC_GRID,),
    in_specs=[
        pl.BlockSpec((1, 1, TC_BLOCK), lambda i: (i, 0, 0)),
        pl.BlockSpec((NUM_TYPES, D), lambda i: (0, 0)),
    ],
    out_specs=pl.BlockSpec((TC_BLOCK, D), lambda i: (i, 0)),
    out_shape=jax.ShapeDtypeStruct((N_NODES, D), jnp.float32),
)


def kernel(atom_types, embed_table):
    idx = atom_types.reshape(-1).astype(jnp.int32)
    out0 = _embed_sc(idx, embed_table)
    out1 = _embed_tc(idx.reshape(TC_GRID, 1, TC_BLOCK), embed_table)
    return (out0, out1)


# speedup vs baseline: 1.6455x; 1.1928x over previous
import functools

import jax
import jax.numpy as jnp
from jax import lax
from jax.experimental import pallas as pl
from jax.experimental.pallas import tpu as pltpu
from jax.experimental.pallas import tpu_sc as plsc

N_NODES = 100000
D = 128
NUM_TYPES = 64

_info = plsc.get_sparse_core_info()
NC, NS = _info.num_cores, _info.num_subcores
NW = NC * NS

MAIN = (N_NODES // (8 * NW)) * (8 * NW)
B_W = MAIN // NW
CHUNK = 104
N_CHUNKS = B_W // CHUNK
NBUF = 5
N_ROUNDS = N_CHUNKS // NBUF
TAIL = N_NODES - MAIN
TAIL_STEP = 8
TAIL_W = TAIL // TAIL_STEP

_mesh = plsc.VectorSubcoreMesh(core_axis_name="c", subcore_axis_name="s")


@functools.partial(
    pl.kernel,
    mesh=_mesh,
    out_type=jax.ShapeDtypeStruct((N_NODES, D), jnp.float32),
    scratch_types=[
        pltpu.VMEM((B_W,), jnp.int32),
        pltpu.VMEM((TAIL_STEP,), jnp.int32),
    ] + [pltpu.VMEM((CHUNK, D), jnp.float32) for _ in range(NBUF)] + [
        pltpu.VMEM_SHARED((NUM_TYPES, D), jnp.float32),
    ] + [pltpu.SemaphoreType.DMA for _ in range(2 * NBUF)],
)
def _embed_sc(idx_hbm, table_hbm, out_hbm, idx_v, tidx_v, *rest):
    bufs = rest[:NBUF]
    table_sh = rest[NBUF]
    gsem = rest[NBUF + 1:NBUF + 1 + NBUF]
    ssem = rest[NBUF + 1 + NBUF:]
    wid = lax.axis_index("s") * NC + lax.axis_index("c")
    base = wid * B_W

    @pl.when(lax.axis_index("s") == 0)
    def _stage():
        pltpu.sync_copy(table_hbm, table_sh)
    plsc.subcore_barrier()

    pltpu.sync_copy(idx_hbm.at[pl.ds(base, B_W)], idx_v)

    def gather_start(c, b):
        pltpu.async_copy(
            table_sh.at[idx_v.at[pl.ds(c * CHUNK, CHUNK)]], bufs[b], gsem[b]
        )

    def gather_wait(b):
        pltpu.make_async_copy(
            table_sh.at[idx_v.at[pl.ds(0, CHUNK)]], bufs[b], gsem[b]
        ).wait()

    def store_start(c, b):
        pltpu.async_copy(
            bufs[b], out_hbm.at[pl.ds(base + c * CHUNK, CHUNK)], ssem[b]
        )

    def store_wait(b):
        pltpu.make_async_copy(
            bufs[b], out_hbm.at[pl.ds(base, CHUNK)], ssem[b]
        ).wait()

    def body(r, carry):
        for b in range(NBUF):
            @pl.when(r > 0)
            def _(b=b):
                store_wait(b)
            gather_start(r * NBUF + b, b)
        for b in range(NBUF):
            gather_wait(b)
            store_start(r * NBUF + b, b)
        return carry

    lax.fori_loop(0, N_ROUNDS, body, 0)
    for b in range(NBUF):
        store_wait(b)

    @pl.when(wid < TAIL_W)
    def _tail():
        tb = MAIN + wid * TAIL_STEP
        pltpu.sync_copy(idx_hbm.at[pl.ds(tb, TAIL_STEP)], tidx_v)
        pltpu.async_copy(
            table_sh.at[tidx_v], bufs[0].at[pl.ds(0, TAIL_STEP)], gsem[0]
        ).wait()
        pltpu.sync_copy(
            bufs[0].at[pl.ds(0, TAIL_STEP)], out_hbm.at[pl.ds(tb, TAIL_STEP)]
        )


TC_BLOCK = 4000
TC_GRID = N_NODES // TC_BLOCK


def _tc_body(idx_ref, table_ref, out_ref):
    idx_row = idx_ref[0]
    tids = lax.broadcasted_iota(jnp.int32, (NUM_TYPES, TC_BLOCK), 0)
    onehot_t = (idx_row == tids).astype(jnp.bfloat16)
    table_bf = table_ref[...].astype(jnp.bfloat16)
    out_ref[...] = lax.dot_general(
        onehot_t, table_bf, (((0,), (0,)), ((), ())),
        preferred_element_type=jnp.float32,
    )


_embed_tc = pl.pallas_call(
    _tc_body,
    grid=(TC_GRID,),
    in_specs=[
        pl.BlockSpec((1, 1, TC_BLOCK), lambda i: (i, 0, 0)),
        pl.BlockSpec((NUM_TYPES, D), lambda i: (0, 0)),
    ],
    out_specs=pl.BlockSpec((TC_BLOCK, D), lambda i: (i, 0)),
    out_shape=jax.ShapeDtypeStruct((N_NODES, D), jnp.float32),
)


def kernel(atom_types, embed_table):
    idx = atom_types.reshape(-1).astype(jnp.int32)
    out0 = _embed_sc(idx, embed_table)
    out1 = _embed_tc(idx.reshape(TC_GRID, 1, TC_BLOCK), embed_table)
    return (out0, out1)


# speedup vs baseline: 1.6805x; 1.0213x over previous
import functools

import jax
import jax.numpy as jnp
from jax import lax
from jax.experimental import pallas as pl
from jax.experimental.pallas import tpu as pltpu
from jax.experimental.pallas import tpu_sc as plsc

N_NODES = 100000
D = 128
NUM_TYPES = 64

_info = plsc.get_sparse_core_info()
NC, NS = _info.num_cores, _info.num_subcores
NW = NC * NS

MAIN = (N_NODES // (8 * NW)) * (8 * NW)
B_W = MAIN // NW
CHUNK = 104
N_CHUNKS = B_W // CHUNK
NBUF = 5
N_ROUNDS = N_CHUNKS // NBUF
TAIL = N_NODES - MAIN
TAIL_STEP = 8
TAIL_W = TAIL // TAIL_STEP

_mesh = plsc.VectorSubcoreMesh(core_axis_name="c", subcore_axis_name="s")


@functools.partial(
    pl.kernel,
    mesh=_mesh,
    out_type=(
        jax.ShapeDtypeStruct((N_NODES, D), jnp.float32),
        jax.ShapeDtypeStruct((N_NODES, D), jnp.float32),
    ),
    scratch_types=[
        pltpu.VMEM((B_W,), jnp.int32),
        pltpu.VMEM((TAIL_STEP,), jnp.int32),
    ] + [pltpu.VMEM((CHUNK, D), jnp.float32) for _ in range(NBUF)] + [
        pltpu.VMEM_SHARED((NUM_TYPES, D), jnp.float32),
    ] + [pltpu.SemaphoreType.DMA for _ in range(2 * NBUF)],
)
def _embed(idx_hbm, table_hbm, out0_hbm, out1_hbm, idx_v, tidx_v, *rest):
    bufs = rest[:NBUF]
    table_sh = rest[NBUF]
    gsem = rest[NBUF + 1:NBUF + 1 + NBUF]
    ssem = rest[NBUF + 1 + NBUF:]
    wid = lax.axis_index("s") * NC + lax.axis_index("c")
    base = wid * B_W

    @pl.when(lax.axis_index("s") == 0)
    def _stage():
        pltpu.sync_copy(table_hbm, table_sh)
    plsc.subcore_barrier()

    pltpu.sync_copy(idx_hbm.at[pl.ds(base, B_W)], idx_v)

    outs = (out0_hbm, out1_hbm)

    def gather_start(c, b):
        pltpu.async_copy(
            table_sh.at[idx_v.at[pl.ds(c * CHUNK, CHUNK)]], bufs[b], gsem[b]
        )

    def gather_wait(b):
        pltpu.make_async_copy(
            table_sh.at[idx_v.at[pl.ds(0, CHUNK)]], bufs[b], gsem[b]
        ).wait()

    def store_start(c, b):
        for o in outs:
            pltpu.async_copy(
                bufs[b], o.at[pl.ds(base + c * CHUNK, CHUNK)], ssem[b]
            )

    def store_wait(b):
        for o in outs:
            pltpu.make_async_copy(
                bufs[b], o.at[pl.ds(base, CHUNK)], ssem[b]
            ).wait()

    def body(r, carry):
        for b in range(NBUF):
            @pl.when(r > 0)
            def _(b=b):
                store_wait(b)
            gather_start(r * NBUF + b, b)
        for b in range(NBUF):
            gather_wait(b)
            store_start(r * NBUF + b, b)
        return carry

    lax.fori_loop(0, N_ROUNDS, body, 0)
    for b in range(NBUF):
        store_wait(b)

    @pl.when(wid < TAIL_W)
    def _tail():
        tb = MAIN + wid * TAIL_STEP
        pltpu.sync_copy(idx_hbm.at[pl.ds(tb, TAIL_STEP)], tidx_v)
        pltpu.async_copy(
            table_sh.at[tidx_v], bufs[0].at[pl.ds(0, TAIL_STEP)], gsem[0]
        ).wait()
        for o in outs:
            pltpu.sync_copy(
                bufs[0].at[pl.ds(0, TAIL_STEP)], o.at[pl.ds(tb, TAIL_STEP)]
            )


def kernel(atom_types, embed_table):
    idx = atom_types.reshape(-1)
    out0, out1 = _embed(idx, embed_table)
    return (out0, out1)


# speedup vs baseline: 1.6863x; 1.0035x over previous
import functools

import jax
import jax.numpy as jnp
from jax import lax
from jax.experimental import pallas as pl
from jax.experimental.pallas import tpu as pltpu
from jax.experimental.pallas import tpu_sc as plsc

N_NODES = 100000
D = 128
NUM_TYPES = 64

_info = plsc.get_sparse_core_info()
NC, NS = _info.num_cores, _info.num_subcores
NW = NC * NS

MAIN = (N_NODES // (8 * NW)) * (8 * NW)
B_W = MAIN // NW
CHUNK = 104
N_CHUNKS = B_W // CHUNK
NBUF = 5
N_ROUNDS = N_CHUNKS // NBUF
TAIL = N_NODES - MAIN
TAIL_STEP = 8
TAIL_W = TAIL // TAIL_STEP

_mesh = plsc.VectorSubcoreMesh(core_axis_name="c", subcore_axis_name="s")


@functools.partial(
    pl.kernel,
    mesh=_mesh,
    out_type=(
        jax.ShapeDtypeStruct((N_NODES, D), jnp.float32),
        jax.ShapeDtypeStruct((N_NODES, D), jnp.float32),
    ),
    scratch_types=[
        pltpu.VMEM((B_W,), jnp.int32),
        pltpu.VMEM((TAIL_STEP,), jnp.int32),
    ] + [pltpu.VMEM((CHUNK, D), jnp.float32) for _ in range(NBUF)] + [
        pltpu.VMEM_SHARED((NUM_TYPES, D), jnp.float32),
    ] + [pltpu.SemaphoreType.DMA for _ in range(2 * NBUF)],
)
def _embed(idx_hbm, table_hbm, out0_hbm, out1_hbm, idx_v, tidx_v, *rest):
    bufs = rest[:NBUF]
    table_sh = rest[NBUF]
    gsem = rest[NBUF + 1:NBUF + 1 + NBUF]
    ssem = rest[NBUF + 1 + NBUF:]
    wid = lax.axis_index("s") * NC + lax.axis_index("c")
    base = wid * B_W

    @pl.when(lax.axis_index("s") == 0)
    def _stage():
        pltpu.sync_copy(table_hbm, table_sh)
    pltpu.sync_copy(idx_hbm.at[pl.ds(base, B_W)], idx_v)
    plsc.subcore_barrier()

    outs = (out0_hbm, out1_hbm)

    def gather_start(c, b):
        pltpu.async_copy(
            table_sh.at[idx_v.at[pl.ds(c * CHUNK, CHUNK)]], bufs[b], gsem[b]
        )

    def gather_wait(b):
        pltpu.make_async_copy(
            table_sh.at[idx_v.at[pl.ds(0, CHUNK)]], bufs[b], gsem[b]
        ).wait()

    def store_start(c, b):
        for o in outs:
            pltpu.async_copy(
                bufs[b], o.at[pl.ds(base + c * CHUNK, CHUNK)], ssem[b]
            )

    def store_wait(b):
        for o in outs:
            pltpu.make_async_copy(
                bufs[b], o.at[pl.ds(base, CHUNK)], ssem[b]
            ).wait()

    def body(r, carry):
        for b in range(NBUF):
            @pl.when(r > 0)
            def _(b=b):
                store_wait(b)
            gather_start(r * NBUF + b, b)
        for b in range(NBUF):
            gather_wait(b)
            store_start(r * NBUF + b, b)
        return carry

    lax.fori_loop(0, N_ROUNDS, body, 0)
    for b in range(NBUF):
        store_wait(b)

    @pl.when(wid < TAIL_W)
    def _tail():
        tb = MAIN + wid * TAIL_STEP
        pltpu.sync_copy(idx_hbm.at[pl.ds(tb, TAIL_STEP)], tidx_v)
        pltpu.async_copy(
            table_sh.at[tidx_v], bufs[0].at[pl.ds(0, TAIL_STEP)], gsem[0]
        ).wait()
        for o in outs:
            pltpu.sync_copy(
                bufs[0].at[pl.ds(0, TAIL_STEP)], o.at[pl.ds(tb, TAIL_STEP)]
            )


def kernel(atom_types, embed_table):
    idx = atom_types.reshape(-1)
    out0, out1 = _embed(idx, embed_table)
    return (out0, out1)
